# baseline (speedup 1.0000x reference)
import functools

import jax
import jax.numpy as jnp
from jax import lax
from jax.experimental import pallas as pl
from jax.experimental.pallas import tpu as pltpu
from jax.experimental.pallas import tpu_sc as plsc

B = 4
N = 512
C = 16
D = 1024
NBITS = 6
R = B * N

SC_TOK = 512

NW = 32
RPW = SC_TOK // NW
ROWB = 2
NBUF = 2
NSTEP = RPW // ROWB
NGRP = D // 16

TOK_BLK = 128
TC_TOK = R - SC_TOK


def _sc_body(x_hbm, ht_hbm, eidx_hbm, conf_hbm,
             xbuf, htbuf, eidx_v, conf_v, sems):
    cid = lax.axis_index("c")
    sid = lax.axis_index("s")
    wid = sid * 2 + cid
    base = wid * RPW

    pltpu.sync_copy(ht_hbm, htbuf)

    lane = jnp.arange(16, dtype=jnp.int32)
    mask2 = lane < 2
    mask1 = lane < 1

    def issue(buf, step):
        pltpu.make_async_copy(
            x_hbm.at[pl.ds((base + step * ROWB) * C, ROWB * C)],
            xbuf.at[buf], sems.at[buf]).start()

    def wait(buf):
        pltpu.make_async_copy(
            x_hbm.at[pl.ds(0, ROWB * C)], xbuf.at[buf], sems.at[buf]).wait()

    for bf in range(NBUF):
        issue(bf, bf)

    def compute(buf, step):
        def gbody(g, paccs):
            col = g * 16
            hts = [htbuf[j, pl.ds(col, 16)] for j in range(NBITS)]
            out = []
            for r in range(ROWB):
                acc = xbuf[buf, r * C, pl.ds(col, 16)]
                for c in range(1, C):
                    acc = acc + xbuf[buf, r * C + c, pl.ds(col, 16)]
                u = plsc.bitcast(acc * (1.0 / C), jnp.int32)
                u = (u + jnp.int32(0x7FFF) +
                     (lax.shift_right_logical(u, jnp.int32(16)) &
                      jnp.int32(1))) & jnp.int32(-65536)
                emb = plsc.bitcast(u, jnp.float32)
                for j in range(NBITS):
                    out.append(paccs[r * NBITS + j] + emb * hts[j])
            return tuple(out)

        z = jnp.zeros((16,), dtype=jnp.float32)
        paccs = lax.fori_loop(0, NGRP, gbody, (z,) * (ROWB * NBITS))

        for r in range(ROWB):
            ps = [jnp.sum(paccs[r * NBITS + j]) for j in range(NBITS)]
            eid = jnp.int32(0)
            for j in range(NBITS):
                eid = eid + jnp.where(ps[j] > 0, jnp.int32(1 << j),
                                      jnp.int32(0))
            aj = [jnp.abs(p) for p in ps]
            m = aj[0]
            flip = jnp.int32(1)
            for j in range(1, NBITS):
                cnd = aj[j] < m
                m = jnp.where(cnd, aj[j], m)
                flip = jnp.where(cnd, jnp.int32(1 << j), flip)
            e2 = lax.bitwise_xor(eid, flip)
            conf = (aj[0] + aj[1] + aj[2] + aj[3] + aj[4] + aj[5]) \
                * (1.0 / NBITS)

            rl = step * ROWB + r
            ev = jnp.where(lane == 0, jnp.full((16,), eid, jnp.int32),
                           jnp.full((16,), e2, jnp.int32))
            plsc.store_scatter(eidx_v, [2 * rl + lane], ev, mask=mask2)
            plsc.store_scatter(conf_v, [rl + lane * 0],
                               jnp.full((16,), conf, jnp.float32),
                               mask=mask1)

    def sbody(s2, carry):
        for bf in range(NBUF):
            step = s2 * NBUF + bf
            wait(bf)
            compute(bf, step)
            nxt = step + NBUF

            @pl.when(nxt < NSTEP)
            def _():
                issue(bf, nxt)
        return carry

    lax.fori_loop(0, NSTEP // NBUF, sbody, jnp.int32(0))

    pltpu.sync_copy(eidx_v, eidx_hbm.at[pl.ds(base * 2, RPW * 2)])
    pltpu.sync_copy(conf_v, conf_hbm.at[pl.ds(base, RPW)])


def _sc_router(x2, ht):
    mesh = plsc.VectorSubcoreMesh(core_axis_name="c", subcore_axis_name="s")
    return pl.kernel(
        _sc_body,
        out_type=[
            jax.ShapeDtypeStruct((SC_TOK * 2,), jnp.int32),
            jax.ShapeDtypeStruct((SC_TOK,), jnp.float32),
        ],
        mesh=mesh,
        scratch_types=[
            pltpu.VMEM((NBUF, ROWB * C, D), jnp.float32),
            pltpu.VMEM((NBITS, D), jnp.float32),
            pltpu.VMEM((RPW * 2,), jnp.int32),
            pltpu.VMEM((RPW,), jnp.float32),
            pltpu.SemaphoreType.DMA((NBUF,)),
        ],
        compiler_params=pltpu.CompilerParams(needs_layout_passes=False),
    )(x2, ht)


def _tc_body(x_ref, h_ref, e1_ref, e2_ref, conf_ref):
    xb = x_ref[...]
    xs = jnp.sum(xb.reshape(TOK_BLK, C, D), axis=1) * (1.0 / C)
    xs16 = xs.astype(jnp.bfloat16)
    h16 = h_ref[...].astype(jnp.bfloat16)
    proj = jax.lax.dot_general(
        xs16, h16, (((1,), (0,)), ((), ())),
        preferred_element_type=jnp.float32)
    lane = lax.broadcasted_iota(jnp.int32, proj.shape, 1)
    valid = lane < NBITS
    pw = jnp.where(valid & (proj > 0), jnp.int32(1) << lane, jnp.int32(0))
    e1 = jnp.sum(pw, axis=1)
    a = jnp.abs(proj)
    am = jnp.where(valid, a, jnp.float32(3.4e38))
    m = jnp.min(am, axis=1, keepdims=True)
    weakest = jnp.min(jnp.where(am == m, lane, jnp.int32(127)), axis=1)
    e2 = lax.bitwise_xor(e1, jnp.int32(1) << weakest)
    conf = jnp.sum(jnp.where(valid, a, 0.0), axis=1) * (1.0 / NBITS)
    e1_ref[...] = e1.reshape(1, 1, TOK_BLK)
    e2_ref[...] = e2.reshape(1, 1, TOK_BLK)
    conf_ref[...] = conf.reshape(1, 1, TOK_BLK)


def _tc_router(x2, hpad):
    nblk = TC_TOK // TOK_BLK
    blk0 = SC_TOK // TOK_BLK
    e1, e2, conf = pl.pallas_call(
        _tc_body,
        grid=(nblk,),
        in_specs=[
            pl.BlockSpec((TOK_BLK * C, D), lambda i: (blk0 + i, 0)),
            pl.BlockSpec((D, 128), lambda i: (0, 0)),
        ],
        out_specs=[
            pl.BlockSpec((1, 1, TOK_BLK), lambda i: (i, 0, 0)),
            pl.BlockSpec((1, 1, TOK_BLK), lambda i: (i, 0, 0)),
            pl.BlockSpec((1, 1, TOK_BLK), lambda i: (i, 0, 0)),
        ],
        out_shape=[
            jax.ShapeDtypeStruct((nblk, 1, TOK_BLK), jnp.int32),
            jax.ShapeDtypeStruct((nblk, 1, TOK_BLK), jnp.int32),
            jax.ShapeDtypeStruct((nblk, 1, TOK_BLK), jnp.float32),
        ],
    )(x2, hpad)
    return e1.reshape(TC_TOK), e2.reshape(TC_TOK), conf.reshape(TC_TOK)


def _round_bf16(v):
    u = lax.bitcast_convert_type(v, jnp.int32)
    u = (u + jnp.int32(0x7FFF) +
         (lax.shift_right_logical(u, 16) & jnp.int32(1))) & jnp.int32(-65536)
    return lax.bitcast_convert_type(u, jnp.float32)


@jax.jit
def _router(x2, ht, hpad):
    tc_e1, tc_e2, tc_conf = _tc_router(x2, hpad)
    sc_eidx, sc_conf = _sc_router(x2, ht)
    eidx = jnp.concatenate(
        [sc_eidx.reshape(SC_TOK, 2),
         jnp.stack([tc_e1, tc_e2], axis=-1)], axis=0)
    conf = jnp.concatenate([sc_conf, tc_conf], axis=0)
    return eidx, conf


def kernel(x, hyperplanes):
    x2 = x.reshape(R * C, D)
    ht = _round_bf16(hyperplanes.T.reshape(NBITS, D))
    hpad = jnp.zeros((D, 128), jnp.float32).at[:, :NBITS].set(hyperplanes)
    eidx, conf = _router(x2, ht, hpad)
    return (eidx.reshape(B, N, 2),
            jnp.ones((B, N, 2), x.dtype),
            conf.reshape(B, N))


# speedup vs baseline: 1.0013x; 1.0013x over previous
import jax
import jax.numpy as jnp
from jax import lax
from jax.experimental import pallas as pl
from jax.experimental.pallas import tpu as pltpu
from jax.experimental.pallas import tpu_sc as plsc

B = 4
N = 512
C = 16
D = 1024
NBITS = 6
R = B * N

SC_TOK = 768

NW = 32
RPW = SC_TOK // NW
ROWB = 2
NBUF = 2
NSTEP = RPW // ROWB
NGRP = D // 16

TOK_BLK = 128
TC_TOK = R - SC_TOK


def _sc_body(x_hbm, ht_hbm, eidx_hbm, conf_hbm,
             xbuf, htbuf, eidx_v, conf_v, sems):
    cid = lax.axis_index("c")
    sid = lax.axis_index("s")
    wid = sid * 2 + cid
    base = wid * RPW

    pltpu.sync_copy(ht_hbm, htbuf)

    lane = jnp.arange(16, dtype=jnp.int32)
    mask2 = lane < 2
    mask1 = lane < 1

    def issue(buf, step):
        pltpu.make_async_copy(
            x_hbm.at[pl.ds((base + step * ROWB) * C, ROWB * C)],
            xbuf.at[buf], sems.at[buf]).start()

    def wait(buf):
        pltpu.make_async_copy(
            x_hbm.at[pl.ds(0, ROWB * C)], xbuf.at[buf], sems.at[buf]).wait()

    for bf in range(NBUF):
        issue(bf, bf)

    def compute(buf, step):
        def gbody(g, paccs):
            col = g * 16
            hts = [htbuf[j, pl.ds(col, 16)] for j in range(NBITS)]
            out = []
            for r in range(ROWB):
                acc = xbuf[buf, r * C, pl.ds(col, 16)]
                for c in range(1, C):
                    acc = acc + xbuf[buf, r * C + c, pl.ds(col, 16)]
                u = plsc.bitcast(acc * (1.0 / C), jnp.int32)
                u = (u + jnp.int32(0x7FFF) +
                     (lax.shift_right_logical(u, jnp.int32(16)) &
                      jnp.int32(1))) & jnp.int32(-65536)
                emb = plsc.bitcast(u, jnp.float32)
                for j in range(NBITS):
                    out.append(paccs[r * NBITS + j] + emb * hts[j])
            return tuple(out)

        z = jnp.zeros((16,), dtype=jnp.float32)
        paccs = lax.fori_loop(0, NGRP, gbody, (z,) * (ROWB * NBITS))

        for r in range(ROWB):
            ps = [jnp.sum(paccs[r * NBITS + j]) for j in range(NBITS)]
            eid = jnp.int32(0)
            for j in range(NBITS):
                eid = eid + jnp.where(ps[j] > 0, jnp.int32(1 << j),
                                      jnp.int32(0))
            aj = [jnp.abs(p) for p in ps]
            m = aj[0]
            flip = jnp.int32(1)
            for j in range(1, NBITS):
                cnd = aj[j] < m
                m = jnp.where(cnd, aj[j], m)
                flip = jnp.where(cnd, jnp.int32(1 << j), flip)
            e2 = lax.bitwise_xor(eid, flip)
            conf = (aj[0] + aj[1] + aj[2] + aj[3] + aj[4] + aj[5]) \
                * (1.0 / NBITS)

            rl = step * ROWB + r
            ev = jnp.where(lane == 0, jnp.full((16,), eid, jnp.int32),
                           jnp.full((16,), e2, jnp.int32))
            plsc.store_scatter(eidx_v, [2 * rl + lane], ev, mask=mask2)
            plsc.store_scatter(conf_v, [rl + lane * 0],
                               jnp.full((16,), conf, jnp.float32),
                               mask=mask1)

    def sbody(s2, carry):
        for bf in range(NBUF):
            step = s2 * NBUF + bf
            wait(bf)
            compute(bf, step)
            nxt = step + NBUF

            @pl.when(nxt < NSTEP)
            def _():
                issue(bf, nxt)
        return carry

    lax.fori_loop(0, NSTEP // NBUF, sbody, jnp.int32(0))

    pltpu.sync_copy(eidx_v, eidx_hbm.at[pl.ds(base * 2, RPW * 2)])
    pltpu.sync_copy(conf_v, conf_hbm.at[pl.ds(base, RPW)])


def _sc_router(x2, ht):
    mesh = plsc.VectorSubcoreMesh(core_axis_name="c", subcore_axis_name="s")
    return pl.kernel(
        _sc_body,
        out_type=[
            jax.ShapeDtypeStruct((SC_TOK * 2,), jnp.int32),
            jax.ShapeDtypeStruct((SC_TOK,), jnp.float32),
        ],
        mesh=mesh,
        scratch_types=[
            pltpu.VMEM((NBUF, ROWB * C, D), jnp.float32),
            pltpu.VMEM((NBITS, D), jnp.float32),
            pltpu.VMEM((RPW * 2,), jnp.int32),
            pltpu.VMEM((RPW,), jnp.float32),
            pltpu.SemaphoreType.DMA((NBUF,)),
        ],
        compiler_params=pltpu.CompilerParams(needs_layout_passes=False),
    )(x2, ht)


def _tc_body(x_ref, h_ref, e1_ref, e2_ref, conf_ref):
    xb = x_ref[...]
    xs = jnp.sum(xb.reshape(TOK_BLK, C, D), axis=1) * (1.0 / C)
    xs16 = xs.astype(jnp.bfloat16)
    h16 = h_ref[...].astype(jnp.bfloat16)
    proj = jax.lax.dot_general(
        xs16, h16, (((1,), (0,)), ((), ())),
        preferred_element_type=jnp.float32)
    lane = lax.broadcasted_iota(jnp.int32, proj.shape, 1)
    valid = lane < NBITS
    pw = jnp.where(valid & (proj > 0), jnp.int32(1) << lane, jnp.int32(0))
    e1 = jnp.sum(pw, axis=1)
    a = jnp.abs(proj)
    am = jnp.where(valid, a, jnp.float32(3.4e38))
    m = jnp.min(am, axis=1, keepdims=True)
    weakest = jnp.min(jnp.where(am == m, lane, jnp.int32(127)), axis=1)
    e2 = lax.bitwise_xor(e1, jnp.int32(1) << weakest)
    conf = jnp.sum(jnp.where(valid, a, 0.0), axis=1) * (1.0 / NBITS)
    e1_ref[...] = e1.reshape(1, 1, TOK_BLK)
    e2_ref[...] = e2.reshape(1, 1, TOK_BLK)
    conf_ref[...] = conf.reshape(1, 1, TOK_BLK)


def _tc_router(x2, hpad):
    nblk = TC_TOK // TOK_BLK
    blk0 = SC_TOK // TOK_BLK
    e1, e2, conf = pl.pallas_call(
        _tc_body,
        grid=(nblk,),
        in_specs=[
            pl.BlockSpec((TOK_BLK * C, D), lambda i: (blk0 + i, 0)),
            pl.BlockSpec((D, 128), lambda i: (0, 0)),
        ],
        out_specs=[
            pl.BlockSpec((1, 1, TOK_BLK), lambda i: (i, 0, 0)),
            pl.BlockSpec((1, 1, TOK_BLK), lambda i: (i, 0, 0)),
            pl.BlockSpec((1, 1, TOK_BLK), lambda i: (i, 0, 0)),
        ],
        out_shape=[
            jax.ShapeDtypeStruct((nblk, 1, TOK_BLK), jnp.int32),
            jax.ShapeDtypeStruct((nblk, 1, TOK_BLK), jnp.int32),
            jax.ShapeDtypeStruct((nblk, 1, TOK_BLK), jnp.float32),
        ],
    )(x2, hpad)
    return e1.reshape(TC_TOK), e2.reshape(TC_TOK), conf.reshape(TC_TOK)


def _round_bf16(v):
    u = lax.bitcast_convert_type(v, jnp.int32)
    u = (u + jnp.int32(0x7FFF) +
         (lax.shift_right_logical(u, 16) & jnp.int32(1))) & jnp.int32(-65536)
    return lax.bitcast_convert_type(u, jnp.float32)


@jax.jit
def _router(x2, ht, hpad):
    tc_e1, tc_e2, tc_conf = _tc_router(x2, hpad)
    sc_eidx, sc_conf = _sc_router(x2, ht)
    eidx = jnp.concatenate(
        [sc_eidx.reshape(SC_TOK, 2),
         jnp.stack([tc_e1, tc_e2], axis=-1)], axis=0)
    conf = jnp.concatenate([sc_conf, tc_conf], axis=0)
    return eidx, conf


def kernel(x, hyperplanes):
    x2 = x.reshape(R * C, D)
    ht = _round_bf16(hyperplanes.T.reshape(NBITS, D))
    hpad = jnp.zeros((D, 128), jnp.float32).at[:, :NBITS].set(hyperplanes)
    eidx, conf = _router(x2, ht, hpad)
    return (eidx.reshape(B, N, 2),
            jnp.ones((B, N, 2), x.dtype),
            conf.reshape(B, N))
